# baseline (speedup 1.0000x reference)
import functools

import jax
import jax.numpy as jnp
from jax.experimental import pallas as pl
from jax.experimental.pallas import tpu as pltpu

_ALPHA = 0.2
_LOG2E = 1.4426950408889634
_BR = 256
_AUG = 128


def _aug(wh, dout):
    br = wh.shape[0]
    return jnp.concatenate(
        [wh, jnp.ones((br, 1), jnp.float32),
         jnp.zeros((br, _AUG - dout - 1), jnp.float32)],
        axis=1).astype(jnp.bfloat16)


def _exp_pair(logits):
    t = logits * jnp.float32(_LOG2E)
    return (jnp.exp2(t).astype(jnp.bfloat16),
            jnp.exp2(t * jnp.float32(_ALPHA)).astype(jnp.bfloat16))


def _exp_w(logits):
    return jnp.exp2(
        logits * jnp.float32(_LOG2E * (_ALPHA - 1.0))).astype(jnp.bfloat16)


def _attn_rows(adjf, wh_aug, w_col, v_row, va_row, colmean):
    p = jnp.maximum(v_row, w_col * va_row) * adjf
    dout = colmean.shape[1]
    o_aug = jnp.dot(p, wh_aug, preferred_element_type=jnp.float32)
    o = o_aug[:, :dout]
    s = o_aug[:, dout:dout + 1]
    rs = jnp.where(s > 0, 1.0 / jnp.where(s > 0, s, 1.0), 0.0)
    z = jnp.where(s > 0, 0.0, 1.0)
    return o * rs + z * colmean


def _head_logits(wh, a_ref, hd, dout, es_s, edt_s, r0, br):
    asrc = a_ref[hd:hd + 1, :dout]
    adst = a_ref[hd:hd + 1, dout:]
    es = jax.lax.dot_general(wh, asrc, (((1,), (1,)), ((), ())),
                             preferred_element_type=jnp.float32)
    es_s[pl.ds(r0, br), hd:hd + 1] = _exp_w(es)
    edt = jax.lax.dot_general(adst, wh, (((1,), (1,)), ((), ())),
                              preferred_element_type=jnp.float32)
    v, va = _exp_pair(edt)
    edt_s[2 * hd:2 * hd + 1, pl.ds(r0, br)] = v
    edt_s[2 * hd + 1:2 * hd + 2, pl.ds(r0, br)] = va


def _accum_colsum(cs_s, i, contrib):
    @pl.when(i == 0)
    def _():
        cs_s[...] = contrib

    @pl.when(i != 0)
    def _():
        cs_s[...] = cs_s[...] + contrib


def _body(x_ref, adj_ref, w1_ref, a1_ref, w2_ref, a2_ref, out_ref,
          adjb_s, wh1_s0, wh1_s1, wh1_s2, wh1_s3, wh2_s0, wh2_s1,
          es1_s, edt1_s, cs1_s, es2_s, edt2_s, cs2_s, *, nheads, nouts,
          dout, nclass, br, n):
    ph = pl.program_id(0)
    i = pl.program_id(1)
    r0 = i * br
    wh1_s = (wh1_s0, wh1_s1, wh1_s2, wh1_s3)
    wh2_s = (wh2_s0, wh2_s1)

    @pl.when(ph == 0)
    def _proj():
        xb = x_ref[...]
        contribs = []
        for h in range(nheads):
            wh = jnp.dot(xb, w1_ref[h], preferred_element_type=jnp.float32)
            wh1_s[h][pl.ds(r0, br), :] = _aug(wh, dout)
            contribs.append(jnp.sum(wh, axis=0, keepdims=True))
            _head_logits(wh, a1_ref, h, dout, es1_s, edt1_s, r0, br)
        _accum_colsum(cs1_s, i, jnp.concatenate(contribs, axis=0))

    @pl.when(ph == 1)
    def _att1():
        adjf = adj_ref[...].astype(jnp.bfloat16)
        adjb_s[pl.ds(r0, br), :] = adjf
        cols = []
        for h in range(nheads):
            oh = _attn_rows(adjf, wh1_s[h][...],
                            es1_s[pl.ds(r0, br), h:h + 1],
                            edt1_s[2 * h:2 * h + 1, :],
                            edt1_s[2 * h + 1:2 * h + 2, :],
                            cs1_s[h:h + 1, :] * (1.0 / n))
            cols.append(
                jnp.where(oh > 0, oh, jnp.exp(jnp.minimum(oh, 0.0)) - 1.0))
        h1b = jnp.concatenate(cols, axis=1)
        contribs = []
        for j in range(nouts):
            whj = jnp.dot(h1b, w2_ref[j], preferred_element_type=jnp.float32)
            wh2_s[j][pl.ds(r0, br), :] = _aug(whj, nclass)
            contribs.append(jnp.sum(whj, axis=0, keepdims=True))
            _head_logits(whj, a2_ref, j, nclass, es2_s, edt2_s, r0, br)
        _accum_colsum(cs2_s, i, jnp.concatenate(contribs, axis=0))

    @pl.when(ph == 2)
    def _att2():
        adjf = adjb_s[pl.ds(r0, br), :]
        acc = None
        for j in range(nouts):
            oj = _attn_rows(adjf, wh2_s[j][...],
                            es2_s[pl.ds(r0, br), j:j + 1],
                            edt2_s[2 * j:2 * j + 1, :],
                            edt2_s[2 * j + 1:2 * j + 2, :],
                            cs2_s[j:j + 1, :] * (1.0 / n))
            acc = oj if acc is None else acc + oj
        o = acc * (1.0 / nouts)
        m = jnp.max(o, axis=1, keepdims=True)
        lse = jnp.log(jnp.sum(jnp.exp(o - m), axis=1, keepdims=True)) + m
        out_ref[...] = o - lse


def kernel(x, adj, W1, a1, W2, a2):
    n, nfeat = x.shape
    nheads, _, dout = W1.shape
    nouts, nhid_tot, nclass = W2.shape
    br = min(_BR, n)

    full = lambda shape: pl.BlockSpec(shape, lambda p, i: (0,) * len(shape))
    out = pl.pallas_call(
        functools.partial(_body, nheads=nheads, nouts=nouts, dout=dout,
                          nclass=nclass, br=br, n=n),
        grid=(3, n // br),
        in_specs=[
            pl.BlockSpec((br, nfeat), lambda p, i: (jnp.where(p == 0, i, 0), 0)),
            pl.BlockSpec((br, n), lambda p, i: (jnp.where(p == 1, i, 0), 0)),
            full((nheads, nfeat, dout)),
            full((nheads, 2 * dout)),
            full((nouts, nhid_tot, nclass)),
            full((nouts, 2 * nclass)),
        ],
        out_specs=pl.BlockSpec((br, nclass), lambda p, i: (i, 0)),
        out_shape=jax.ShapeDtypeStruct((n, nclass), jnp.float32),
        scratch_shapes=[
            pltpu.VMEM((n, n), jnp.bfloat16),
        ] + [pltpu.VMEM((n, _AUG), jnp.bfloat16)] * nheads
          + [pltpu.VMEM((n, _AUG), jnp.bfloat16)] * nouts
          + [
            pltpu.VMEM((n, nheads), jnp.bfloat16),
            pltpu.VMEM((2 * nheads, n), jnp.bfloat16),
            pltpu.VMEM((nheads, dout), jnp.float32),
            pltpu.VMEM((n, nouts), jnp.bfloat16),
            pltpu.VMEM((2 * nouts, n), jnp.bfloat16),
            pltpu.VMEM((nouts, nclass), jnp.float32),
        ],
    )
    return out(x, adj, W1, a1, W2, a2)


# speedup vs baseline: 1.1706x; 1.1706x over previous
import functools

import jax
import jax.numpy as jnp
from jax.experimental import pallas as pl
from jax.experimental.pallas import tpu as pltpu

_ALPHA = 0.2
_LOG2E = 1.4426950408889634
_BR = 512
_AUG = 128


def _aug(wh, dout):
    br = wh.shape[0]
    return jnp.concatenate(
        [wh, jnp.ones((br, 1), jnp.float32),
         jnp.zeros((br, _AUG - dout - 1), jnp.float32)],
        axis=1).astype(jnp.bfloat16)


def _exp_pair(logits):
    t = logits * jnp.float32(_LOG2E)
    return (jnp.exp2(t).astype(jnp.bfloat16),
            jnp.exp2(t * jnp.float32(_ALPHA)).astype(jnp.bfloat16))


def _exp_w(logits):
    return jnp.exp2(
        logits * jnp.float32(_LOG2E * (_ALPHA - 1.0))).astype(jnp.bfloat16)


def _attn_rows(adjf, wh_aug, w_col, v_row, va_row, colmean):
    p = jnp.maximum(v_row, w_col * va_row) * adjf
    dout = colmean.shape[1]
    o_aug = jnp.dot(p, wh_aug, preferred_element_type=jnp.float32)
    o = o_aug[:, :dout]
    s = o_aug[:, dout:dout + 1]
    rs = jnp.where(s > 0, 1.0 / jnp.where(s > 0, s, 1.0), 0.0)
    z = jnp.where(s > 0, 0.0, 1.0)
    return o * rs + z * colmean


def _head_logits(wh, a_ref, hd, dout, es_s, edt_s, r0, br):
    asrc = a_ref[hd:hd + 1, :dout]
    adst = a_ref[hd:hd + 1, dout:]
    es = jax.lax.dot_general(wh, asrc, (((1,), (1,)), ((), ())),
                             preferred_element_type=jnp.float32)
    es_s[pl.ds(r0, br), hd:hd + 1] = _exp_w(es)
    edt = jax.lax.dot_general(adst, wh, (((1,), (1,)), ((), ())),
                              preferred_element_type=jnp.float32)
    v, va = _exp_pair(edt)
    edt_s[2 * hd:2 * hd + 1, pl.ds(r0, br)] = v
    edt_s[2 * hd + 1:2 * hd + 2, pl.ds(r0, br)] = va


def _accum_colsum(cs_s, i, contrib):
    @pl.when(i == 0)
    def _():
        cs_s[...] = contrib

    @pl.when(i != 0)
    def _():
        cs_s[...] = cs_s[...] + contrib


def _body(x_ref, adj_ref, w1_ref, a1_ref, w2_ref, a2_ref, out_ref,
          adjb_s, wh1_s0, wh1_s1, wh1_s2, wh1_s3, wh2_s0, wh2_s1,
          es1_s, edt1_s, cs1_s, es2_s, edt2_s, cs2_s, *, nheads, nouts,
          dout, nclass, br, n):
    ph = pl.program_id(0)
    i = pl.program_id(1)
    r0 = i * br
    wh1_s = (wh1_s0, wh1_s1, wh1_s2, wh1_s3)
    wh2_s = (wh2_s0, wh2_s1)

    @pl.when(ph == 0)
    def _proj():
        xb = x_ref[...]
        contribs = []
        for h in range(nheads):
            wh = jnp.dot(xb, w1_ref[h], preferred_element_type=jnp.float32)
            wh1_s[h][pl.ds(r0, br), :] = _aug(wh, dout)
            contribs.append(jnp.sum(wh, axis=0, keepdims=True))
            _head_logits(wh, a1_ref, h, dout, es1_s, edt1_s, r0, br)
        _accum_colsum(cs1_s, i, jnp.concatenate(contribs, axis=0))

    @pl.when(ph == 1)
    def _att1():
        adjf = adj_ref[...].astype(jnp.bfloat16)
        adjb_s[pl.ds(r0, br), :] = adjf.astype(jnp.float8_e5m2)
        cols = []
        for h in range(nheads):
            oh = _attn_rows(adjf, wh1_s[h][...],
                            es1_s[pl.ds(r0, br), h:h + 1],
                            edt1_s[2 * h:2 * h + 1, :],
                            edt1_s[2 * h + 1:2 * h + 2, :],
                            cs1_s[h:h + 1, :] * (1.0 / n))
            cols.append(
                jnp.where(oh > 0, oh, jnp.exp(jnp.minimum(oh, 0.0)) - 1.0))
        h1b = jnp.concatenate(cols, axis=1)
        contribs = []
        for j in range(nouts):
            whj = jnp.dot(h1b, w2_ref[j], preferred_element_type=jnp.float32)
            wh2_s[j][pl.ds(r0, br), :] = _aug(whj, nclass)
            contribs.append(jnp.sum(whj, axis=0, keepdims=True))
            _head_logits(whj, a2_ref, j, nclass, es2_s, edt2_s, r0, br)
        _accum_colsum(cs2_s, i, jnp.concatenate(contribs, axis=0))

    @pl.when(ph == 2)
    def _att2():
        adjf = adjb_s[pl.ds(r0, br), :].astype(jnp.bfloat16)
        acc = None
        for j in range(nouts):
            oj = _attn_rows(adjf, wh2_s[j][...],
                            es2_s[pl.ds(r0, br), j:j + 1],
                            edt2_s[2 * j:2 * j + 1, :],
                            edt2_s[2 * j + 1:2 * j + 2, :],
                            cs2_s[j:j + 1, :] * (1.0 / n))
            acc = oj if acc is None else acc + oj
        o = acc * (1.0 / nouts)
        m = jnp.max(o, axis=1, keepdims=True)
        lse = jnp.log(jnp.sum(jnp.exp(o - m), axis=1, keepdims=True)) + m
        out_ref[...] = o - lse


def kernel(x, adj, W1, a1, W2, a2):
    n, nfeat = x.shape
    nheads, _, dout = W1.shape
    nouts, nhid_tot, nclass = W2.shape
    br = min(_BR, n)

    full = lambda shape: pl.BlockSpec(shape, lambda p, i: (0,) * len(shape))
    out = pl.pallas_call(
        functools.partial(_body, nheads=nheads, nouts=nouts, dout=dout,
                          nclass=nclass, br=br, n=n),
        grid=(3, n // br),
        in_specs=[
            pl.BlockSpec((br, nfeat), lambda p, i: (jnp.where(p == 0, i, 0), 0)),
            pl.BlockSpec((br, n), lambda p, i: (jnp.where(p == 1, i, 0), 0)),
            full((nheads, nfeat, dout)),
            full((nheads, 2 * dout)),
            full((nouts, nhid_tot, nclass)),
            full((nouts, 2 * nclass)),
        ],
        out_specs=pl.BlockSpec((br, nclass), lambda p, i: (i, 0)),
        out_shape=jax.ShapeDtypeStruct((n, nclass), jnp.float32),
        scratch_shapes=[
            pltpu.VMEM((n, n), jnp.float8_e5m2),
        ] + [pltpu.VMEM((n, _AUG), jnp.bfloat16)] * nheads
          + [pltpu.VMEM((n, _AUG), jnp.bfloat16)] * nouts
          + [
            pltpu.VMEM((n, nheads), jnp.bfloat16),
            pltpu.VMEM((2 * nheads, n), jnp.bfloat16),
            pltpu.VMEM((nheads, dout), jnp.float32),
            pltpu.VMEM((n, nouts), jnp.bfloat16),
            pltpu.VMEM((2 * nouts, n), jnp.bfloat16),
            pltpu.VMEM((nouts, nclass), jnp.float32),
        ],
    )
    return out(x, adj, W1, a1, W2, a2)
